# baseline (speedup 1.0000x reference)
import functools

import jax
import jax.numpy as jnp
from jax import lax
from jax.experimental import pallas as pl
from jax.experimental.pallas import tpu as pltpu
from jax.experimental.pallas import tpu_sc as plsc

_VOCAB = 1000000
_D = 32
_B = 4096
_L = 200
_N = _B * _L
_NW = 32
_PER_W = _N // _NW
_CHUNK = 1600
_NCHUNK = _PER_W // _CHUNK

_mesh = plsc.VectorSubcoreMesh(core_axis_name="c", subcore_axis_name="s")


_WBLK = 4096


def _eye(n):
    return (lax.broadcasted_iota(jnp.int32, (n, n), 0)
            == lax.broadcasted_iota(jnp.int32, (n, n), 1)).astype(jnp.float32)


def _wt_body(wt_ref, out_ref):
    x = wt_ref[...]
    xt = lax.dot_general(x, _eye(_D), (((0,), (0,)), ((), ())),
                         precision=lax.Precision.HIGHEST)
    y = xt.reshape(_WBLK // 32, 8, 4, _D)
    out_ref[...] = jnp.concatenate([y[:, :, e, :] for e in range(4)],
                                   axis=-1)


_wt_transpose = pl.pallas_call(
    _wt_body,
    grid=(pl.cdiv(_VOCAB, _WBLK),),
    in_specs=[pl.BlockSpec((_D, _WBLK), lambda g: (0, g))],
    out_specs=pl.BlockSpec((_WBLK // 32, 8, 128), lambda g: (g, 0, 0)),
    out_shape=jax.ShapeDtypeStruct((_VOCAB * _D // 1024, 8, 128),
                                   jnp.float32),
)


@functools.partial(
    pl.kernel,
    out_type=jax.ShapeDtypeStruct((_N, _D), jnp.float32),
    mesh=_mesh,
    scratch_types=[
        pltpu.VMEM((_CHUNK,), jnp.int32),
        pltpu.VMEM((_CHUNK,), jnp.int32),
        pltpu.VMEM((_CHUNK, _D), jnp.float32),
        pltpu.VMEM((_CHUNK, _D), jnp.float32),
        pltpu.SemaphoreType.DMA,
        pltpu.SemaphoreType.DMA,
        pltpu.SemaphoreType.DMA,
        pltpu.SemaphoreType.DMA,
        pltpu.SemaphoreType.DMA,
        pltpu.SemaphoreType.DMA,
    ],
    compiler_params=pltpu.CompilerParams(use_tc_tiling_on_sc=False),
)
def _emb_lookup(idx_hbm, table_hbm, out_hbm,
                idx0, idx1, rows0, rows1,
                si0, si1, sg0, sg1, so0, so1):
    wid = lax.axis_index("s") * 2 + lax.axis_index("c")
    base = wid * _PER_W

    idx_bufs = (idx0, idx1)
    row_bufs = (rows0, rows1)
    isems = (si0, si1)
    gsems = (sg0, sg1)
    osems = (so0, so1)

    def idx_copy(i):
        b = i % 2
        return pltpu.make_async_copy(
            idx_hbm.at[pl.ds(base + i * _CHUNK, _CHUNK)], idx_bufs[b], isems[b])

    def gather_copy(i):
        b = i % 2
        return pltpu.make_async_copy(table_hbm.at[idx_bufs[b]], row_bufs[b],
                                     gsems[b])

    def out_copy(i):
        b = i % 2
        return pltpu.make_async_copy(
            row_bufs[b], out_hbm.at[pl.ds(base + i * _CHUNK, _CHUNK)], osems[b])

    idx_copy(0).start()
    idx_copy(1).start()
    for i in range(_NCHUNK):
        idx_copy(i).wait()
        if i >= 2:
            out_copy(i - 2).wait()
        gather_copy(i).start()
        gather_copy(i).wait()
        if i + 2 < _NCHUNK:
            idx_copy(i + 2).start()
        out_copy(i).start()
    out_copy(_NCHUNK - 2).wait()
    out_copy(_NCHUNK - 1).wait()


_UNIT = 1024
_NUNITS = _N // _UNIT


def _out_body(g_ref, out_ref):
    x = g_ref[...].reshape(128, 2, 128)
    eye = _eye(128)
    dn = (((1,), (1,)), ((), ()))
    te = lax.dot_general(eye, x[:, 0, :], dn,
                         precision=lax.Precision.HIGHEST)
    to = lax.dot_general(eye, x[:, 1, :], dn,
                         precision=lax.Precision.HIGHEST)
    z = jnp.concatenate([te.reshape(4, _D, 128), to.reshape(4, _D, 128)],
                        axis=0)
    out_ref[...] = z.reshape(32, 1, 8, 128)


_NBT = _B // 128


_out_transpose = pl.pallas_call(
    _out_body,
    grid=(_NUNITS,),
    in_specs=[pl.BlockSpec((256, 128), lambda g: (g, 0))],
    out_specs=pl.BlockSpec((32, 1, 8, 128),
                           lambda g: (g // _NBT, g % _NBT, 0, 0)),
    out_shape=jax.ShapeDtypeStruct((_L * 4, _NBT, 8, 128), jnp.float32),
)


def kernel(input_ids, weight):
    ids_perm = (input_ids.T.reshape(_L // 8, 8, _B // 128, 128)
                .transpose(0, 2, 3, 1).reshape(-1).astype(jnp.int32))
    table = _wt_transpose(weight.T).reshape(_VOCAB, _D)
    rows = _emb_lookup(ids_perm, table)
    out3 = _out_transpose(rows.reshape(_N * _D // 128, 128))
    out = (out3.reshape(_L, _D // 8, _NBT, 8, 128)
           .transpose(2, 4, 0, 1, 3).reshape(_B, _L, _D))
    return out


# speedup vs baseline: 1.5737x; 1.5737x over previous
import functools

import jax
import jax.numpy as jnp
from jax import lax
from jax.experimental import pallas as pl
from jax.experimental.pallas import tpu as pltpu
from jax.experimental.pallas import tpu_sc as plsc

_VOCAB = 1000000
_D = 32
_B = 4096
_L = 200
_N = _B * _L
_NW = 32
_PER_W = _N // _NW
_CHUNK = 1600
_NCHUNK = _PER_W // _CHUNK

_mesh = plsc.VectorSubcoreMesh(core_axis_name="c", subcore_axis_name="s")


_WBLK = 4096


def _eye(n):
    return (lax.broadcasted_iota(jnp.int32, (n, n), 0)
            == lax.broadcasted_iota(jnp.int32, (n, n), 1)).astype(jnp.float32)


def _wt_body(wt_ref, out_ref):
    x = wt_ref[...]
    xt = jnp.swapaxes(x, 0, 1)
    y = xt.reshape(_WBLK // 32, 8, 4, _D)
    out_ref[...] = jnp.concatenate([y[:, :, e, :] for e in range(4)],
                                   axis=-1)


_wt_transpose = pl.pallas_call(
    _wt_body,
    grid=(pl.cdiv(_VOCAB, _WBLK),),
    in_specs=[pl.BlockSpec((_D, _WBLK), lambda g: (0, g))],
    out_specs=pl.BlockSpec((_WBLK // 32, 8, 128), lambda g: (g, 0, 0)),
    out_shape=jax.ShapeDtypeStruct((_VOCAB * _D // 1024, 8, 128),
                                   jnp.float32),
)


@functools.partial(
    pl.kernel,
    out_type=jax.ShapeDtypeStruct((_N, _D), jnp.float32),
    mesh=_mesh,
    scratch_types=[
        pltpu.VMEM((_CHUNK,), jnp.int32),
        pltpu.VMEM((_CHUNK,), jnp.int32),
        pltpu.VMEM((_CHUNK, _D), jnp.float32),
        pltpu.VMEM((_CHUNK, _D), jnp.float32),
        pltpu.SemaphoreType.DMA,
        pltpu.SemaphoreType.DMA,
        pltpu.SemaphoreType.DMA,
        pltpu.SemaphoreType.DMA,
        pltpu.SemaphoreType.DMA,
        pltpu.SemaphoreType.DMA,
    ],
    compiler_params=pltpu.CompilerParams(use_tc_tiling_on_sc=False),
)
def _emb_lookup(idx_hbm, table_hbm, out_hbm,
                idx0, idx1, rows0, rows1,
                si0, si1, sg0, sg1, so0, so1):
    wid = lax.axis_index("s") * 2 + lax.axis_index("c")
    base = wid * _PER_W

    idx_bufs = (idx0, idx1)
    row_bufs = (rows0, rows1)
    isems = (si0, si1)
    gsems = (sg0, sg1)
    osems = (so0, so1)

    def idx_copy(i):
        b = i % 2
        return pltpu.make_async_copy(
            idx_hbm.at[pl.ds(base + i * _CHUNK, _CHUNK)], idx_bufs[b], isems[b])

    def gather_copy(i):
        b = i % 2
        return pltpu.make_async_copy(table_hbm.at[idx_bufs[b]], row_bufs[b],
                                     gsems[b])

    def out_copy(i):
        b = i % 2
        return pltpu.make_async_copy(
            row_bufs[b], out_hbm.at[pl.ds(base + i * _CHUNK, _CHUNK)], osems[b])

    idx_copy(0).start()
    idx_copy(1).start()
    for i in range(_NCHUNK):
        idx_copy(i).wait()
        if i >= 2:
            out_copy(i - 2).wait()
        gather_copy(i).start()
        gather_copy(i).wait()
        if i + 2 < _NCHUNK:
            idx_copy(i + 2).start()
        out_copy(i).start()
    out_copy(_NCHUNK - 2).wait()
    out_copy(_NCHUNK - 1).wait()


_UNIT = 1024
_NUNITS = _N // _UNIT


def _out_body(g_ref, out_ref):
    x = g_ref[...].reshape(128, 2, 128)
    eye = _eye(128)
    dn = (((1,), (1,)), ((), ()))
    te = lax.dot_general(eye, x[:, 0, :], dn,
                         precision=lax.Precision.HIGHEST)
    to = lax.dot_general(eye, x[:, 1, :], dn,
                         precision=lax.Precision.HIGHEST)
    z = jnp.concatenate([te.reshape(4, _D, 128), to.reshape(4, _D, 128)],
                        axis=0)
    out_ref[...] = z.reshape(32, 1, 8, 128)


_NBT = _B // 128


_out_transpose = pl.pallas_call(
    _out_body,
    grid=(_NUNITS,),
    in_specs=[pl.BlockSpec((256, 128), lambda g: (g, 0))],
    out_specs=pl.BlockSpec((32, 1, 8, 128),
                           lambda g: (g // _NBT, g % _NBT, 0, 0)),
    out_shape=jax.ShapeDtypeStruct((_L * 4, _NBT, 8, 128), jnp.float32),
)


def kernel(input_ids, weight):
    flat = input_ids.reshape(-1).astype(jnp.int32)
    table = _wt_transpose(weight.T).reshape(_VOCAB, _D)
    rows = _emb_lookup(flat, table)
    return rows.reshape(input_ids.shape + (weight.shape[1],))


# speedup vs baseline: 1.5973x; 1.0150x over previous
import functools

import jax
import jax.numpy as jnp
from jax import lax
from jax.experimental import pallas as pl
from jax.experimental.pallas import tpu as pltpu
from jax.experimental.pallas import tpu_sc as plsc

_VOCAB = 1000000
_D = 32
_B = 4096
_L = 200
_N = _B * _L
_NW = 32
_PER_W = _N // _NW
_CHUNK = 1600
_NCHUNK = _PER_W // _CHUNK

_mesh = plsc.VectorSubcoreMesh(core_axis_name="c", subcore_axis_name="s")


_WBLK = 16384


def _eye(n):
    return (lax.broadcasted_iota(jnp.int32, (n, n), 0)
            == lax.broadcasted_iota(jnp.int32, (n, n), 1)).astype(jnp.float32)


def _wt_body(wt_ref, out_ref):
    x = wt_ref[...]
    xt = jnp.swapaxes(x, 0, 1)
    y = xt.reshape(_WBLK // 32, 8, 4, _D)
    out_ref[...] = jnp.concatenate([y[:, :, e, :] for e in range(4)],
                                   axis=-1)


_wt_transpose = pl.pallas_call(
    _wt_body,
    grid=(pl.cdiv(_VOCAB, _WBLK),),
    in_specs=[pl.BlockSpec((_D, _WBLK), lambda g: (0, g))],
    out_specs=pl.BlockSpec((_WBLK // 32, 8, 128), lambda g: (g, 0, 0)),
    out_shape=jax.ShapeDtypeStruct((_VOCAB * _D // 1024, 8, 128),
                                   jnp.float32),
)


@functools.partial(
    pl.kernel,
    out_type=jax.ShapeDtypeStruct((_N, _D), jnp.float32),
    mesh=_mesh,
    scratch_types=[
        pltpu.VMEM((_CHUNK,), jnp.int32),
        pltpu.VMEM((_CHUNK,), jnp.int32),
        pltpu.VMEM((_CHUNK, _D), jnp.float32),
        pltpu.VMEM((_CHUNK, _D), jnp.float32),
        pltpu.SemaphoreType.DMA,
        pltpu.SemaphoreType.DMA,
        pltpu.SemaphoreType.DMA,
        pltpu.SemaphoreType.DMA,
        pltpu.SemaphoreType.DMA,
        pltpu.SemaphoreType.DMA,
    ],
    compiler_params=pltpu.CompilerParams(use_tc_tiling_on_sc=False),
)
def _emb_lookup(idx_hbm, table_hbm, out_hbm,
                idx0, idx1, rows0, rows1,
                si0, si1, sg0, sg1, so0, so1):
    wid = lax.axis_index("s") * 2 + lax.axis_index("c")
    base = wid * _PER_W

    idx_bufs = (idx0, idx1)
    row_bufs = (rows0, rows1)
    isems = (si0, si1)
    gsems = (sg0, sg1)
    osems = (so0, so1)

    def idx_copy(i):
        b = i % 2
        return pltpu.make_async_copy(
            idx_hbm.at[pl.ds(base + i * _CHUNK, _CHUNK)], idx_bufs[b], isems[b])

    def gather_copy(i):
        b = i % 2
        return pltpu.make_async_copy(table_hbm.at[idx_bufs[b]], row_bufs[b],
                                     gsems[b])

    def out_copy(i):
        b = i % 2
        return pltpu.make_async_copy(
            row_bufs[b], out_hbm.at[pl.ds(base + i * _CHUNK, _CHUNK)], osems[b])

    idx_copy(0).start()
    idx_copy(1).start()
    for i in range(_NCHUNK):
        idx_copy(i).wait()
        if i >= 2:
            out_copy(i - 2).wait()
        gather_copy(i).start()
        gather_copy(i).wait()
        if i + 2 < _NCHUNK:
            idx_copy(i + 2).start()
        out_copy(i).start()
    out_copy(_NCHUNK - 2).wait()
    out_copy(_NCHUNK - 1).wait()


_UNIT = 1024
_NUNITS = _N // _UNIT


def _out_body(g_ref, out_ref):
    x = g_ref[...].reshape(128, 2, 128)
    eye = _eye(128)
    dn = (((1,), (1,)), ((), ()))
    te = lax.dot_general(eye, x[:, 0, :], dn,
                         precision=lax.Precision.HIGHEST)
    to = lax.dot_general(eye, x[:, 1, :], dn,
                         precision=lax.Precision.HIGHEST)
    z = jnp.concatenate([te.reshape(4, _D, 128), to.reshape(4, _D, 128)],
                        axis=0)
    out_ref[...] = z.reshape(32, 1, 8, 128)


_NBT = _B // 128


_out_transpose = pl.pallas_call(
    _out_body,
    grid=(_NUNITS,),
    in_specs=[pl.BlockSpec((256, 128), lambda g: (g, 0))],
    out_specs=pl.BlockSpec((32, 1, 8, 128),
                           lambda g: (g // _NBT, g % _NBT, 0, 0)),
    out_shape=jax.ShapeDtypeStruct((_L * 4, _NBT, 8, 128), jnp.float32),
)


def kernel(input_ids, weight):
    flat = input_ids.reshape(-1).astype(jnp.int32)
    table = _wt_transpose(weight.T).reshape(_VOCAB, _D)
    rows = _emb_lookup(flat, table)
    return rows.reshape(input_ids.shape + (weight.shape[1],))
